# Initial kernel scaffold; baseline (speedup 1.0000x reference)
import jax
import jax.numpy as jnp
from jax.experimental import pallas as pl


def kernel(x, x_table, y_table, z_table):
    raise NotImplementedError("write your pallas kernel here")


# speedup vs baseline: 3.0772x; 3.0772x over previous
import functools

import jax
import jax.numpy as jnp
from jax.experimental import pallas as pl


def _pos_kernel(xt_ref, yt_ref, zt_ref, out_ref, *, hb: int):
    p = pl.program_id(1)
    ib = pl.program_id(2)
    _, _, _, w, d, f = out_ref.shape

    def fx():
        rows = xt_ref[pl.ds(ib * hb, hb), :]
        out_ref[0, 0] = jnp.broadcast_to(rows[:, None, None, :], (hb, w, d, f))

    def fy():
        rows = yt_ref[:, :]
        out_ref[0, 0] = jnp.broadcast_to(rows[None, :, None, :], (hb, w, d, f))

    def fz():
        rows = zt_ref[:, :]
        out_ref[0, 0] = jnp.broadcast_to(rows[None, None, :, :], (hb, w, d, f))

    jax.lax.switch(p, [fx, fy, fz])


@jax.jit
def kernel(x, x_table, y_table, z_table):
    bs, _, h, w, d = x.shape
    f = x_table.shape[-1]
    hb = 8
    grid = (bs, 3, h // hb)

    tbl_spec = pl.BlockSpec((h, f), lambda b, p, i: (0, 0))
    out_spec = pl.BlockSpec(
        (1, 1, hb, w, d, f), lambda b, p, i: (b, p, i, 0, 0, 0)
    )
    return pl.pallas_call(
        functools.partial(_pos_kernel, hb=hb),
        grid=grid,
        in_specs=[tbl_spec, tbl_spec, tbl_spec],
        out_specs=out_spec,
        out_shape=jax.ShapeDtypeStruct((bs, 3, h, w, d, f), x_table.dtype),
    )(x_table, y_table, z_table)
